# Initial kernel scaffold; baseline (speedup 1.0000x reference)
import jax
import jax.numpy as jnp
from jax.experimental import pallas as pl


def kernel(h, edge_index, W1, a1, W2, a2):
    raise NotImplementedError("write your pallas kernel here")


# speedup vs baseline: 34.8593x; 34.8593x over previous
import functools

import jax
import jax.numpy as jnp
import numpy as np
from jax import lax
from jax.experimental import pallas as pl
from jax.experimental.pallas import tpu as pltpu
from jax.experimental.pallas import tpu_sc as plsc

F32 = jnp.float32
I32 = jnp.int32


def _leaky(x):
  return jnp.where(x >= 0, x, 0.01 * x)


_GATHER_DNUMS = lax.GatherDimensionNumbers(
    offset_dims=(), collapsed_slice_dims=(0,), start_index_map=(0,))


def _lane_gather(v, idx):
  return lax.gather(v, idx[:, None], _GATHER_DNUMS, (1,),
                    mode=lax.GatherScatterMode.PROMISE_IN_BOUNDS)


def _tc_prep1(h, w1cat, a_left, a_right):
  n = h.shape[0]

  def body(h_ref, w_ref, al_ref, ar_ref, sta, dta, stb, dtb):
    z = jnp.dot(h_ref[...], w_ref[...], preferred_element_type=F32)
    el = jnp.dot(z, al_ref[...], preferred_element_type=F32)
    er = jnp.dot(z, ar_ref[...], preferred_element_type=F32)
    m = jnp.maximum(jnp.max(el, axis=0, keepdims=True), 0.0)
    c = _leaky(m + er)
    z4 = jnp.zeros((n, 4), F32)
    z56 = jnp.zeros((n, 56), F32)
    z112 = jnp.zeros((n, 112), F32)
    sta[...] = jnp.concatenate([z[:, 0:64], el[:, 0:4], z4, z56], axis=1)
    stb[...] = jnp.concatenate([z[:, 64:128], el[:, 4:8], z4, z56], axis=1)
    dta[...] = jnp.concatenate([er[:, 0:4], z4, c[:, 0:4], z4, z112], axis=1)
    dtb[...] = jnp.concatenate([er[:, 4:8], z4, c[:, 4:8], z4, z112], axis=1)

  st = jax.ShapeDtypeStruct((n, 128), F32)
  return pl.pallas_call(
      body, out_shape=[st, st, st, st])(h, w1cat, a_left, a_right)


def _tc_mid(pa, pb, w2, a2l, a2r, sda, sdb, n):
  def body(pa_ref, pb_ref, w_ref, al_ref, ar_ref, sda_ref, sdb_ref,
           st_ref, dt_ref):
    numa = pa_ref[0, 0:n, 0:64] + pa_ref[1, 0:n, 0:64]
    numb = pb_ref[0, 0:n, 0:64] + pb_ref[1, 0:n, 0:64]
    dena = pa_ref[0, 0:n, 64:80] + pa_ref[1, 0:n, 64:80]
    denb16 = pb_ref[0, 0:n, 64:80] + pb_ref[1, 0:n, 64:80]
    num = jnp.concatenate([numa, numb], axis=1)
    denb = (jnp.dot(dena, sda_ref[...], preferred_element_type=F32)
            + jnp.dot(denb16, sdb_ref[...], preferred_element_type=F32))
    h1 = num / jnp.where(denb > 0, denb, 1.0)
    h1 = jnp.where(h1 > 0, h1, jnp.exp(h1) - 1.0)
    z2 = jnp.dot(h1, w_ref[...], preferred_element_type=F32)
    el2 = jnp.dot(z2, al_ref[...], preferred_element_type=F32)
    er2 = jnp.dot(z2, ar_ref[...], preferred_element_type=F32)
    m2 = jnp.maximum(jnp.max(el2, axis=0, keepdims=True), 0.0)
    c2 = _leaky(m2 + er2)
    st_ref[...] = jnp.concatenate(
        [z2, el2, jnp.zeros((n, 56), F32)], axis=1)
    dt_ref[...] = jnp.concatenate(
        [er2, c2, jnp.zeros((n, 112), F32)], axis=1)

  st = jax.ShapeDtypeStruct((n, 128), F32)
  return pl.pallas_call(
      body, out_shape=[st, st])(pa, pb, w2, a2l, a2r, sda, sdb)


def _tc_final(pz, sden2, n):
  def body(pz_ref, s_ref, out_ref):
    num = pz_ref[0, 0:n, 0:64] + pz_ref[1, 0:n, 0:64]
    den16 = pz_ref[0, 0:n, 64:80] + pz_ref[1, 0:n, 64:80]
    denb = jnp.dot(den16, s_ref[...], preferred_element_type=F32)
    out_ref[...] = num / jnp.where(denb > 0, denb, 1.0)

  return pl.pallas_call(
      body,
      out_shape=jax.ShapeDtypeStruct((n, 64), F32),
  )(pz, sden2)


@functools.lru_cache(maxsize=None)
def _make_edge_kernel(n_nodes, n_edges, n_heads):
  d_pay = 64
  k = 80
  tile_e = n_edges // 32
  nch = tile_e // k
  rows_t = (-(-n_nodes // 16) + 127) // 128 * 128
  n_acc = rows_t * 16
  zch = 128
  nz = rows_t // zch
  npay = d_pay // 16
  assert tile_e % k == 0 and rows_t % zch == 0

  mesh = plsc.VectorSubcoreMesh(core_axis_name="c", subcore_axis_name="s")

  @functools.partial(
      pl.kernel,
      out_type=jax.ShapeDtypeStruct((2, n_acc, 128), F32),
      mesh=mesh,
      scratch_types=[
          pltpu.VMEM((k,), I32),
          pltpu.VMEM((k,), I32),
          pltpu.VMEM((k, 128), F32),
          pltpu.VMEM((k, 128), F32),
          pltpu.VMEM((k, 128), F32),
          pltpu.VMEM((zch, 128), F32),
          pltpu.VMEM_SHARED((n_acc, 128), F32),
          pltpu.SemaphoreType.DMA,
          pltpu.SemaphoreType.DMA,
      ],
  )
  def body(src_hbm, dst_hbm, st_hbm, dt_hbm, out_hbm,
           sidx, didx, srows, drows, stg, zbuf, acc, sem1, sem2):
    cid = lax.axis_index("c")
    sid = lax.axis_index("s")
    lanes = lax.iota(I32, 16)
    idx_c = (lanes & 7) + 8
    zeros16 = jnp.zeros((16,), F32)

    def zrow(r, carry):
      for j in range(8):
        zbuf[r, pl.ds(j * 16, 16)] = zeros16
      return carry

    lax.fori_loop(0, zch, zrow, 0)
    r0 = sid * rows_t
    for jz in range(nz):
      pltpu.sync_copy(zbuf, acc.at[pl.ds(r0 + jz * zch, zch)])
    plsc.subcore_barrier()

    e0 = (cid * 16 + sid) * tile_e

    def chunk(ci, carry):
      base = e0 + ci * k
      pltpu.sync_copy(src_hbm.at[pl.ds(base, k)], sidx)
      pltpu.sync_copy(dst_hbm.at[pl.ds(base, k)], didx)
      cp1 = pltpu.async_copy(st_hbm.at[sidx], srows, sem1)
      cp2 = pltpu.async_copy(dt_hbm.at[didx], drows, sem2)
      cp1.wait()
      cp2.wait()

      def edge(e, ecarry):
        dt = drows[e, pl.ds(0, 16)]
        cv = _lane_gather(dt, idx_c)
        tail = srows[e, pl.ds(d_pay, 16)]
        x = tail + dt
        y = jnp.where(x >= 0, x, 0.01 * x)
        exv = jnp.exp(y - cv)
        exv = jnp.where(lanes < n_heads, exv, 0.0)
        for i in range(npay):
          hd = i * n_heads // npay
          sc = _lane_gather(exv, jnp.full((16,), hd, I32))
          stg[e, pl.ds(i * 16, 16)] = sc * srows[e, pl.ds(i * 16, 16)]
        stg[e, pl.ds(d_pay, 16)] = exv
        for j in range(npay + 1, 8):
          stg[e, pl.ds(j * 16, 16)] = zeros16
        return ecarry

      lax.fori_loop(0, k, edge, 0)
      pltpu.sync_copy(stg, acc.at[didx], add=True)
      return carry

    lax.fori_loop(0, nch, chunk, 0)
    plsc.subcore_barrier()

    for jz in range(nz):
      rr = r0 + jz * zch
      pltpu.sync_copy(acc.at[pl.ds(rr, zch)], zbuf)
      pltpu.sync_copy(zbuf, out_hbm.at[cid, pl.ds(rr, zch)])

  return body


def kernel(h, edge_index, W1, a1, W2, a2):
  n, in_dim = h.shape
  n_heads, _, hid = W1.shape
  out_dim = W2.shape[1]
  src = edge_index[0]
  dst = edge_index[1]

  w1cat = jnp.transpose(W1, (1, 0, 2)).reshape(in_dim, n_heads * hid)
  a_l = a1[:, :hid, 0]
  a_r = a1[:, hid:, 0]
  eye = jnp.eye(n_heads, dtype=F32)
  a_left = (eye[:, None, :] * a_l[:, :, None]).reshape(n_heads * hid, n_heads)
  a_right = (eye[:, None, :] * a_r[:, :, None]).reshape(n_heads * hid, n_heads)
  a2l = jnp.zeros((out_dim, 8), F32).at[:, 0].set(a2[:out_dim, 0])
  a2r = jnp.zeros((out_dim, 8), F32).at[:, 0].set(a2[out_dim:, 0])

  def den_spread(col0):
    s = np.zeros((16, 128), np.float32)
    for hh in range(4):
      s[hh, col0 + 16 * hh: col0 + 16 * (hh + 1)] = 1.0
    return jnp.asarray(s)

  sda = den_spread(0)
  sdb = den_spread(64)
  s2 = np.zeros((16, 64), np.float32)
  s2[0, :] = 1.0
  sden2 = jnp.asarray(s2)

  sta, dta, stb, dtb = _tc_prep1(h, w1cat, a_left, a_right)
  edge4 = _make_edge_kernel(n, src.shape[0], 4)
  pa = edge4(src, dst, sta, dta)
  pb = edge4(src, dst, stb, dtb)
  st2, dt2 = _tc_mid(pa, pb, W2, a2l, a2r, sda, sdb, n)
  pz2 = _make_edge_kernel(n, src.shape[0], 1)(src, dst, st2, dt2)
  return _tc_final(pz2, sden2, n)


# speedup vs baseline: 37.0972x; 1.0642x over previous
import functools

import jax
import jax.numpy as jnp
import numpy as np
from jax import lax
from jax.experimental import pallas as pl
from jax.experimental.pallas import tpu as pltpu
from jax.experimental.pallas import tpu_sc as plsc

F32 = jnp.float32
I32 = jnp.int32


def _leaky(x):
  return jnp.where(x >= 0, x, 0.01 * x)


_GATHER_DNUMS = lax.GatherDimensionNumbers(
    offset_dims=(), collapsed_slice_dims=(0,), start_index_map=(0,))


def _lane_gather(v, idx):
  return lax.gather(v, idx[:, None], _GATHER_DNUMS, (1,),
                    mode=lax.GatherScatterMode.PROMISE_IN_BOUNDS)


def _tc_prep1(h, w1cat, a_left, a_right):
  n = h.shape[0]

  def body(h_ref, w_ref, al_ref, ar_ref, sta, dta, stb, dtb):
    z = jnp.dot(h_ref[...], w_ref[...], preferred_element_type=F32)
    el = jnp.dot(z, al_ref[...], preferred_element_type=F32)
    er = jnp.dot(z, ar_ref[...], preferred_element_type=F32)
    m = jnp.maximum(jnp.max(el, axis=0, keepdims=True), 0.0)
    c = _leaky(m + er)
    z4 = jnp.zeros((n, 4), F32)
    z56 = jnp.zeros((n, 56), F32)
    z112 = jnp.zeros((n, 112), F32)
    sta[...] = jnp.concatenate([z[:, 0:64], el[:, 0:4], z4, z56], axis=1)
    stb[...] = jnp.concatenate([z[:, 64:128], el[:, 4:8], z4, z56], axis=1)
    dta[...] = jnp.concatenate([er[:, 0:4], z4, c[:, 0:4], z4, z112], axis=1)
    dtb[...] = jnp.concatenate([er[:, 4:8], z4, c[:, 4:8], z4, z112], axis=1)

  st = jax.ShapeDtypeStruct((n, 128), F32)
  return pl.pallas_call(
      body, out_shape=[st, st, st, st])(h, w1cat, a_left, a_right)


def _tc_mid(pa, pb, w2, a2l, a2r, sda, sdb, n):
  def body(pa_ref, pb_ref, w_ref, al_ref, ar_ref, sda_ref, sdb_ref,
           st_ref, dt_ref):
    numa = pa_ref[0, 0:n, 0:64] + pa_ref[1, 0:n, 0:64]
    numb = pb_ref[0, 0:n, 0:64] + pb_ref[1, 0:n, 0:64]
    dena = pa_ref[0, 0:n, 64:80] + pa_ref[1, 0:n, 64:80]
    denb16 = pb_ref[0, 0:n, 64:80] + pb_ref[1, 0:n, 64:80]
    num = jnp.concatenate([numa, numb], axis=1)
    denb = (jnp.dot(dena, sda_ref[...], preferred_element_type=F32)
            + jnp.dot(denb16, sdb_ref[...], preferred_element_type=F32))
    h1 = num / jnp.where(denb > 0, denb, 1.0)
    h1 = jnp.where(h1 > 0, h1, jnp.exp(h1) - 1.0)
    z2 = jnp.dot(h1, w_ref[...], preferred_element_type=F32)
    el2 = jnp.dot(z2, al_ref[...], preferred_element_type=F32)
    er2 = jnp.dot(z2, ar_ref[...], preferred_element_type=F32)
    m2 = jnp.maximum(jnp.max(el2, axis=0, keepdims=True), 0.0)
    c2 = _leaky(m2 + er2)
    st_ref[...] = jnp.concatenate(
        [z2, el2, jnp.zeros((n, 56), F32)], axis=1)
    dt_ref[...] = jnp.concatenate(
        [er2, c2, jnp.zeros((n, 112), F32)], axis=1)

  st = jax.ShapeDtypeStruct((n, 128), F32)
  return pl.pallas_call(
      body, out_shape=[st, st])(pa, pb, w2, a2l, a2r, sda, sdb)


def _tc_final(pz, sden2, n):
  def body(pz_ref, s_ref, out_ref):
    num = pz_ref[0, 0:n, 0:64] + pz_ref[1, 0:n, 0:64]
    den16 = pz_ref[0, 0:n, 64:80] + pz_ref[1, 0:n, 64:80]
    denb = jnp.dot(den16, s_ref[...], preferred_element_type=F32)
    out_ref[...] = num / jnp.where(denb > 0, denb, 1.0)

  return pl.pallas_call(
      body,
      out_shape=jax.ShapeDtypeStruct((n, 64), F32),
  )(pz, sden2)


@functools.lru_cache(maxsize=None)
def _make_edge_kernel(n_nodes, n_edges, n_heads):
  d_pay = 64
  k = 80
  tile_e = n_edges // 32
  nch = tile_e // k
  rows_t = (-(-n_nodes // 16) + 127) // 128 * 128
  n_acc = rows_t * 16
  zch = k
  nz = rows_t // zch
  npay = d_pay // 16
  assert nch % 2 == 1 and tile_e % k == 0 and rows_t % zch == 0

  mesh = plsc.VectorSubcoreMesh(core_axis_name="c", subcore_axis_name="s")

  @functools.partial(
      pl.kernel,
      out_type=jax.ShapeDtypeStruct((2, n_acc, 128), F32),
      mesh=mesh,
      scratch_types=[
          pltpu.VMEM((2, k), I32),
          pltpu.VMEM((2, k), I32),
          pltpu.VMEM((k, 128), F32),
          pltpu.VMEM((k, 128), F32),
          pltpu.VMEM((k, 128), F32),
          pltpu.VMEM((k, 128), F32),
          pltpu.VMEM_SHARED((n_acc, 128), F32),
          pltpu.SemaphoreType.DMA,
          pltpu.SemaphoreType.DMA,
          pltpu.SemaphoreType.DMA,
          pltpu.SemaphoreType.DMA,
          pltpu.SemaphoreType.DMA,
          pltpu.SemaphoreType.DMA,
      ],
  )
  def body(src_hbm, dst_hbm, st_hbm, dt_hbm, out_hbm,
           sidx, didx, sr0, dr0, sr1, dr1, acc,
           gs0, gd0, gs1, gd1, sc0, sc1):
    cid = lax.axis_index("c")
    sid = lax.axis_index("s")
    lanes = lax.iota(I32, 16)
    zeros16 = jnp.zeros((16,), F32)
    sets = ((sidx.at[0], didx.at[0], sr0, dr0, gs0, gd0, sc0),
            (sidx.at[1], didx.at[1], sr1, dr1, gs1, gd1, sc1))

    def zrow(r, carry):
      for j in range(8):
        sr0[r, pl.ds(j * 16, 16)] = zeros16
      return carry

    lax.fori_loop(0, zch, zrow, 0)
    r0 = sid * rows_t
    for jz in range(nz):
      pltpu.sync_copy(sr0, acc.at[pl.ds(r0 + jz * zch, zch)])
    plsc.subcore_barrier()

    e0 = (cid * 16 + sid) * tile_e

    def issue(s, base):
      si, di, sr, dr, gs, gd, _ = s
      pltpu.sync_copy(src_hbm.at[pl.ds(base, k)], si)
      pltpu.sync_copy(dst_hbm.at[pl.ds(base, k)], di)
      pltpu.async_copy(st_hbm.at[si], sr, gs)
      pltpu.async_copy(dt_hbm.at[di], dr, gd)

    def process(s):
      si, di, sr, dr, gs, gd, sc = s
      pltpu.make_async_copy(st_hbm.at[si], sr, gs).wait()
      pltpu.make_async_copy(dt_hbm.at[di], dr, gd).wait()

      def edge(e, ecarry):
        dt = dr[e, pl.ds(0, 16)]
        cv = dr[e, pl.ds(8, 16)]
        tail = sr[e, pl.ds(d_pay, 16)]
        x = tail + dt
        y = jnp.where(x >= 0, x, 0.01 * x)
        exv = jnp.exp(y - cv)
        exv = jnp.where(lanes < n_heads, exv, 0.0)
        for i in range(npay):
          hd = i * n_heads // npay
          w = _lane_gather(exv, jnp.full((16,), hd, I32))
          sr[e, pl.ds(i * 16, 16)] = w * sr[e, pl.ds(i * 16, 16)]
        sr[e, pl.ds(d_pay, 16)] = exv
        return ecarry

      lax.fori_loop(0, k, edge, 0, unroll=2)
      pltpu.async_copy(sr, acc.at[di], sc, add=True)

    def drain_scatter(s):
      si, di, sr, dr, gs, gd, sc = s
      pltpu.make_async_copy(sr, acc.at[di], sc).wait()

    issue(sets[0], e0)

    def pair(cj, carry):
      c0 = 2 * cj

      @pl.when(cj > 0)
      def _():
        drain_scatter(sets[1])

      issue(sets[1], e0 + (c0 + 1) * k)
      process(sets[0])
      drain_scatter(sets[0])
      issue(sets[0], e0 + (c0 + 2) * k)
      process(sets[1])
      return carry

    lax.fori_loop(0, (nch - 1) // 2, pair, 0)
    drain_scatter(sets[1])
    process(sets[0])
    drain_scatter(sets[0])
    plsc.subcore_barrier()

    for jz in range(nz):
      rr = r0 + jz * zch
      pltpu.sync_copy(acc.at[pl.ds(rr, zch)], sr0)
      pltpu.sync_copy(sr0, out_hbm.at[cid, pl.ds(rr, zch)])

  return body


def kernel(h, edge_index, W1, a1, W2, a2):
  n, in_dim = h.shape
  n_heads, _, hid = W1.shape
  out_dim = W2.shape[1]
  src = edge_index[0]
  dst = edge_index[1]

  w1cat = jnp.transpose(W1, (1, 0, 2)).reshape(in_dim, n_heads * hid)
  a_l = a1[:, :hid, 0]
  a_r = a1[:, hid:, 0]
  eye = jnp.eye(n_heads, dtype=F32)
  a_left = (eye[:, None, :] * a_l[:, :, None]).reshape(n_heads * hid, n_heads)
  a_right = (eye[:, None, :] * a_r[:, :, None]).reshape(n_heads * hid, n_heads)
  a2l = jnp.zeros((out_dim, 8), F32).at[:, 0].set(a2[:out_dim, 0])
  a2r = jnp.zeros((out_dim, 8), F32).at[:, 0].set(a2[out_dim:, 0])

  def den_spread(col0):
    s = np.zeros((16, 128), np.float32)
    for hh in range(4):
      s[hh, col0 + 16 * hh: col0 + 16 * (hh + 1)] = 1.0
    return jnp.asarray(s)

  sda = den_spread(0)
  sdb = den_spread(64)
  s2 = np.zeros((16, 64), np.float32)
  s2[0, :] = 1.0
  sden2 = jnp.asarray(s2)

  sta, dta, stb, dtb = _tc_prep1(h, w1cat, a_left, a_right)
  edge4 = _make_edge_kernel(n, src.shape[0], 4)
  pa = edge4(src, dst, sta, dta)
  pb = edge4(src, dst, stb, dtb)
  st2, dt2 = _tc_mid(pa, pb, W2, a2l, a2r, sda, sdb, n)
  pz2 = _make_edge_kernel(n, src.shape[0], 1)(src, dst, st2, dt2)
  return _tc_final(pz2, sden2, n)


# speedup vs baseline: 42.2744x; 1.1396x over previous
import functools

import jax
import jax.numpy as jnp
import numpy as np
from jax import lax
from jax.experimental import pallas as pl
from jax.experimental.pallas import tpu as pltpu
from jax.experimental.pallas import tpu_sc as plsc

F32 = jnp.float32
I32 = jnp.int32


def _leaky(x):
  return jnp.where(x >= 0, x, 0.01 * x)


_GATHER_DNUMS = lax.GatherDimensionNumbers(
    offset_dims=(), collapsed_slice_dims=(0,), start_index_map=(0,))


def _lane_gather(v, idx):
  return lax.gather(v, idx[:, None], _GATHER_DNUMS, (1,),
                    mode=lax.GatherScatterMode.PROMISE_IN_BOUNDS)


def _tc_prep1(h, w1cat, a_left, a_right):
  n = h.shape[0]

  def body(h_ref, w_ref, al_ref, ar_ref, sta, dta, stb, dtb):
    z = jnp.dot(h_ref[...], w_ref[...], preferred_element_type=F32)
    el = jnp.dot(z, al_ref[...], preferred_element_type=F32)
    er = jnp.dot(z, ar_ref[...], preferred_element_type=F32)
    m = jnp.maximum(jnp.max(el, axis=0, keepdims=True), 0.0)
    c = _leaky(m + er)
    z4 = jnp.zeros((n, 4), F32)
    z56 = jnp.zeros((n, 56), F32)
    z112 = jnp.zeros((n, 112), F32)
    sta[...] = jnp.concatenate([z[:, 0:64], el[:, 0:4], z4, z56], axis=1)
    stb[...] = jnp.concatenate([z[:, 64:128], el[:, 4:8], z4, z56], axis=1)
    dta[...] = jnp.concatenate([er[:, 0:4], z4, c[:, 0:4], z4, z112], axis=1)
    dtb[...] = jnp.concatenate([er[:, 4:8], z4, c[:, 4:8], z4, z112], axis=1)

  st = jax.ShapeDtypeStruct((n, 128), F32)
  return pl.pallas_call(
      body, out_shape=[st, st, st, st])(h, w1cat, a_left, a_right)


def _tc_mid(pa, pb, w2, a2l, a2r, sda, sdb, n):
  def body(pa_ref, pb_ref, w_ref, al_ref, ar_ref, sda_ref, sdb_ref,
           st_ref, dt_ref):
    numa = pa_ref[0, 0:n, 0:64] + pa_ref[1, 0:n, 0:64]
    numb = pb_ref[0, 0:n, 0:64] + pb_ref[1, 0:n, 0:64]
    dena = pa_ref[0, 0:n, 64:80] + pa_ref[1, 0:n, 64:80]
    denb16 = pb_ref[0, 0:n, 64:80] + pb_ref[1, 0:n, 64:80]
    num = jnp.concatenate([numa, numb], axis=1)
    denb = (jnp.dot(dena, sda_ref[...], preferred_element_type=F32)
            + jnp.dot(denb16, sdb_ref[...], preferred_element_type=F32))
    h1 = num / jnp.where(denb > 0, denb, 1.0)
    h1 = jnp.where(h1 > 0, h1, jnp.exp(h1) - 1.0)
    z2 = jnp.dot(h1, w_ref[...], preferred_element_type=F32)
    el2 = jnp.dot(z2, al_ref[...], preferred_element_type=F32)
    er2 = jnp.dot(z2, ar_ref[...], preferred_element_type=F32)
    m2 = jnp.maximum(jnp.max(el2, axis=0, keepdims=True), 0.0)
    c2 = _leaky(m2 + er2)
    st_ref[...] = jnp.concatenate(
        [z2, el2, jnp.zeros((n, 56), F32)], axis=1)
    dt_ref[...] = jnp.concatenate(
        [er2, c2, jnp.zeros((n, 112), F32)], axis=1)

  st = jax.ShapeDtypeStruct((n, 128), F32)
  return pl.pallas_call(
      body, out_shape=[st, st])(pa, pb, w2, a2l, a2r, sda, sdb)


def _tc_final(pz, sden2, n):
  def body(pz_ref, s_ref, out_ref):
    num = pz_ref[0, 0:n, 0:64] + pz_ref[1, 0:n, 0:64]
    den16 = pz_ref[0, 0:n, 64:80] + pz_ref[1, 0:n, 64:80]
    denb = jnp.dot(den16, s_ref[...], preferred_element_type=F32)
    out_ref[...] = num / jnp.where(denb > 0, denb, 1.0)

  return pl.pallas_call(
      body,
      out_shape=jax.ShapeDtypeStruct((n, 64), F32),
  )(pz, sden2)


@functools.lru_cache(maxsize=None)
def _make_edge_kernel(n_nodes, n_edges, n_heads):
  d_pay = 64
  k = 80
  tile_e = n_edges // 32
  nch = tile_e // k
  rows_t = (-(-n_nodes // 16) + 127) // 128 * 128
  n_acc = rows_t * 16
  zch = k
  nz = rows_t // zch
  npay = d_pay // 16
  assert nch % 2 == 1 and tile_e % k == 0 and rows_t % zch == 0

  mesh = plsc.VectorSubcoreMesh(core_axis_name="c", subcore_axis_name="s")

  @functools.partial(
      pl.kernel,
      out_type=jax.ShapeDtypeStruct((2, n_acc, 128), F32),
      mesh=mesh,
      scratch_types=[
          pltpu.VMEM((2, 2, k), I32),
          pltpu.VMEM((k, 128), F32),
          pltpu.VMEM((k, 128), F32),
          pltpu.VMEM((k, 128), F32),
          pltpu.VMEM((k, 128), F32),
          pltpu.VMEM_SHARED((n_acc, 128), F32),
          pltpu.SemaphoreType.DMA,
          pltpu.SemaphoreType.DMA,
          pltpu.SemaphoreType.DMA,
          pltpu.SemaphoreType.DMA,
          pltpu.SemaphoreType.DMA,
          pltpu.SemaphoreType.DMA,
      ],
  )
  def body(eidx_hbm, st_hbm, dt_hbm, out_hbm,
           idxb, sr0, dr0, sr1, dr1, acc,
           gs0, gd0, gs1, gd1, sc0, sc1):
    cid = lax.axis_index("c")
    sid = lax.axis_index("s")
    lanes = lax.iota(I32, 16)
    zeros16 = jnp.zeros((16,), F32)
    sets = ((idxb.at[0], idxb.at[0, 0], idxb.at[0, 1], sr0, dr0, gs0, gd0, sc0),
            (idxb.at[1], idxb.at[1, 0], idxb.at[1, 1], sr1, dr1, gs1, gd1, sc1))

    def zrow(r, carry):
      for j in range(8):
        sr0[r, pl.ds(j * 16, 16)] = zeros16
      return carry

    lax.fori_loop(0, zch, zrow, 0)
    r0 = sid * rows_t
    for jz in range(nz):
      pltpu.sync_copy(sr0, acc.at[pl.ds(r0 + jz * zch, zch)])
    plsc.subcore_barrier()

    c0glob = (cid * 16 + sid) * nch

    def issue(s, ci):
      both, si, di, sr, dr, gs, gd, _ = s
      pltpu.sync_copy(eidx_hbm.at[c0glob + ci], both)
      pltpu.async_copy(st_hbm.at[si], sr, gs)
      pltpu.async_copy(dt_hbm.at[di], dr, gd)

    def process(s):
      both, si, di, sr, dr, gs, gd, sc = s
      pltpu.make_async_copy(st_hbm.at[si], sr, gs).wait()
      pltpu.make_async_copy(dt_hbm.at[di], dr, gd).wait()

      def edge(e, ecarry):
        dt = dr[e, pl.ds(0, 16)]
        cv = dr[e, pl.ds(8, 16)]
        tail = sr[e, pl.ds(d_pay, 16)]
        x = tail + dt
        y = jnp.where(x >= 0, x, 0.01 * x)
        exv = jnp.exp(y - cv)
        exv = jnp.where(lanes < n_heads, exv, 0.0)
        for i in range(npay):
          hd = i * n_heads // npay
          w = _lane_gather(exv, jnp.full((16,), hd, I32))
          sr[e, pl.ds(i * 16, 16)] = w * sr[e, pl.ds(i * 16, 16)]
        sr[e, pl.ds(d_pay, 16)] = exv
        return ecarry

      lax.fori_loop(0, k, edge, 0, unroll=2)
      pltpu.async_copy(sr, acc.at[di], sc, add=True)

    def drain_scatter(s):
      both, si, di, sr, dr, gs, gd, sc = s
      pltpu.make_async_copy(sr, acc.at[di], sc).wait()

    issue(sets[0], 0)

    def pair(cj, carry):
      c0 = 2 * cj

      @pl.when(cj > 0)
      def _():
        drain_scatter(sets[1])

      issue(sets[1], c0 + 1)
      process(sets[0])
      drain_scatter(sets[0])
      issue(sets[0], c0 + 2)
      process(sets[1])
      return carry

    lax.fori_loop(0, (nch - 1) // 2, pair, 0)
    drain_scatter(sets[1])
    process(sets[0])
    drain_scatter(sets[0])
    plsc.subcore_barrier()

    for jz in range(nz):
      rr = r0 + jz * zch
      pltpu.sync_copy(acc.at[pl.ds(rr, zch)], sr0)
      pltpu.sync_copy(sr0, out_hbm.at[cid, pl.ds(rr, zch)])

  return body


def kernel(h, edge_index, W1, a1, W2, a2):
  n, in_dim = h.shape
  n_heads, _, hid = W1.shape
  out_dim = W2.shape[1]
  src = edge_index[0]
  dst = edge_index[1]

  w1cat = jnp.transpose(W1, (1, 0, 2)).reshape(in_dim, n_heads * hid)
  a_l = a1[:, :hid, 0]
  a_r = a1[:, hid:, 0]
  eye = jnp.eye(n_heads, dtype=F32)
  a_left = (eye[:, None, :] * a_l[:, :, None]).reshape(n_heads * hid, n_heads)
  a_right = (eye[:, None, :] * a_r[:, :, None]).reshape(n_heads * hid, n_heads)
  a2l = jnp.zeros((out_dim, 8), F32).at[:, 0].set(a2[:out_dim, 0])
  a2r = jnp.zeros((out_dim, 8), F32).at[:, 0].set(a2[out_dim:, 0])

  def den_spread(col0):
    s = np.zeros((16, 128), np.float32)
    for hh in range(4):
      s[hh, col0 + 16 * hh: col0 + 16 * (hh + 1)] = 1.0
    return jnp.asarray(s)

  sda = den_spread(0)
  sdb = den_spread(64)
  s2 = np.zeros((16, 64), np.float32)
  s2[0, :] = 1.0
  sden2 = jnp.asarray(s2)

  kk = 80
  eidx = jnp.stack([src.reshape(-1, kk), dst.reshape(-1, kk)], axis=1)

  sta, dta, stb, dtb = _tc_prep1(h, w1cat, a_left, a_right)
  edge4 = _make_edge_kernel(n, src.shape[0], 4)
  pa = edge4(eidx, sta, dta)
  pb = edge4(eidx, stb, dtb)
  st2, dt2 = _tc_mid(pa, pb, W2, a2l, a2r, sda, sdb, n)
  pz2 = _make_edge_kernel(n, src.shape[0], 1)(eidx, st2, dt2)
  return _tc_final(pz2, sden2, n)
